# Initial kernel scaffold; baseline (speedup 1.0000x reference)
import functools

import jax
import jax.numpy as jnp
from jax import lax
from jax.experimental import pallas as pl
from jax.experimental.pallas import tpu as pltpu
from jax.experimental.pallas import tpu_sc as plsc

N_NODES = 10000
E_EDGES = 320000
D = 128
B = 64
FP_DIM = 1024
N_CELLS = 100

NW = 32
K = 128
NCH = 79
E_PAD = NW * NCH * K
DEN_PAD = 10240


def _tc_pre_body(emb_ref, wg_ref, as_ref, ad_ref, ctl_ref, wctl_ref,
                 h_ref, hs_ref, hd_ref, ctlp_ref):
    h = jnp.dot(emb_ref[...], wg_ref[...],
                preferred_element_type=jnp.float32,
                precision=lax.Precision.HIGHEST)
    h_ref[...] = h
    hs_ref[...] = jnp.sum(h * as_ref[...], axis=1, keepdims=True)
    hd_ref[...] = jnp.sum(h * ad_ref[...], axis=1, keepdims=True)
    ctlp_ref[...] = jnp.dot(ctl_ref[...], wctl_ref[...],
                            preferred_element_type=jnp.float32,
                            precision=lax.Precision.HIGHEST)


def _tc_pre(node_emb, W_g, a_s, a_d, ctl, W_ctl):
    return pl.pallas_call(
        _tc_pre_body,
        out_shape=(
            jax.ShapeDtypeStruct((N_NODES, D), jnp.float32),
            jax.ShapeDtypeStruct((N_NODES, 1), jnp.float32),
            jax.ShapeDtypeStruct((N_NODES, 1), jnp.float32),
            jax.ShapeDtypeStruct((B, D), jnp.float32),
        ),
    )(node_emb, W_g, a_s.reshape(1, D), a_d.reshape(1, D), ctl, W_ctl)


def _sc_kernel_body(h_hbm, src_hbm, dst_hbm, w_hbm, hs_hbm, hd_hbm,
                    fp_table_hbm, drug_fp_hbm,
                    agg_out, den_out, fp_out,
                    hs_v, hd_v, src_v, dst_v, w_v, ee_v, rows_v,
                    zden_v, fpi_v, fp_rows_v, agg_sh, den_sh, sem):
    cid = lax.axis_index("c")
    sid = lax.axis_index("s")
    wid = cid * 16 + sid

    z16f = jnp.zeros((16,), jnp.float32)

    def _zrow(r, carry):
        for d8 in range(8):
            rows_v[r, pl.ds(16 * d8, 16)] = z16f
        return carry
    lax.fori_loop(0, 125, _zrow, 0)

    def _zden(i, carry):
        zden_v[pl.ds(i * 16, 16)] = z16f
        return carry
    lax.fori_loop(0, 40, _zden, 0)

    base = sid * 625
    for k in range(5):
        pltpu.sync_copy(rows_v.at[pl.ds(0, 125)],
                        agg_sh.at[pl.ds(base + 125 * k, 125)])
    pltpu.sync_copy(zden_v, den_sh.at[pl.ds(sid * 640, 640)])

    pltpu.sync_copy(hs_hbm, hs_v)
    pltpu.sync_copy(hd_hbm, hd_v)
    start = wid * NCH
    pltpu.sync_copy(src_hbm.at[pl.ds(start, NCH)], src_v)
    pltpu.sync_copy(dst_hbm.at[pl.ds(start, NCH)], dst_v)
    pltpu.sync_copy(w_hbm.at[pl.ds(start, NCH)], w_v)

    @pl.when(wid < 8)
    def _fp_gather():
        pltpu.sync_copy(drug_fp_hbm.at[pl.ds(wid * 8, 8)], fpi_v)
        pltpu.async_copy(fp_table_hbm.at[fpi_v], fp_rows_v, sem).wait()
        pltpu.sync_copy(fp_rows_v, fp_out.at[pl.ds(wid * 8, 8)])

    plsc.subcore_barrier()

    def _chunk(ch, carry):
        pltpu.async_copy(h_hbm.at[src_v.at[ch]], rows_v, sem).wait()
        for g in range(8):
            s16 = src_v[ch, pl.ds(16 * g, 16)]
            d16 = dst_v[ch, pl.ds(16 * g, 16)]
            w16 = w_v[ch, pl.ds(16 * g, 16)]
            a = plsc.load_gather(hs_v, [s16])
            b = plsc.load_gather(hd_v, [d16])
            x = a + b
            e = jnp.where(x >= 0, x, x * jnp.float32(0.2))
            ee_v[pl.ds(16 * g, 16)] = jnp.exp(e) * w16

        def _scale(g, carry2):
            for j in range(16):
                r = g * 16 + j
                sp = plsc.load_gather(
                    ee_v, [jnp.broadcast_to(r.astype(jnp.int32), (16,))])
                for d8 in range(8):
                    sl = pl.ds(16 * d8, 16)
                    rows_v[r, sl] = rows_v[r, sl] * sp
            return carry2
        lax.fori_loop(0, 8, _scale, 0)

        pltpu.sync_copy(rows_v, agg_sh.at[dst_v.at[ch]], add=True)
        pltpu.sync_copy(ee_v, den_sh.at[dst_v.at[ch]], add=True)
        return carry

    lax.fori_loop(0, NCH, _chunk, 0)

    plsc.subcore_barrier()

    @pl.when(sid == 0)
    def _copy_out():
        pltpu.sync_copy(agg_sh, agg_out.at[cid])
        pltpu.sync_copy(den_sh, den_out.at[cid])


def _sc_edge(h, srcp, dstp, wp, hs, hd, fp_table, drug_fp):
    mesh = plsc.VectorSubcoreMesh(core_axis_name="c", subcore_axis_name="s")
    kern = pl.kernel(
        _sc_kernel_body, mesh=mesh,
        out_type=(
            jax.ShapeDtypeStruct((2, N_NODES, D), jnp.float32),
            jax.ShapeDtypeStruct((2, DEN_PAD), jnp.float32),
            jax.ShapeDtypeStruct((B, FP_DIM), jnp.float32),
        ),
        scratch_types=[
            pltpu.VMEM((N_NODES,), jnp.float32),
            pltpu.VMEM((N_NODES,), jnp.float32),
            pltpu.VMEM((NCH, K), jnp.int32),
            pltpu.VMEM((NCH, K), jnp.int32),
            pltpu.VMEM((NCH, K), jnp.float32),
            pltpu.VMEM((K,), jnp.float32),
            pltpu.VMEM((K, D), jnp.float32),
            pltpu.VMEM((640,), jnp.float32),
            pltpu.VMEM((8,), jnp.int32),
            pltpu.VMEM((8, FP_DIM), jnp.float32),
            pltpu.VMEM_SHARED((N_NODES, D), jnp.float32),
            pltpu.VMEM_SHARED((DEN_PAD,), jnp.float32),
            pltpu.SemaphoreType.DMA,
        ],
    )
    return kern(h, srcp, dstp, wp, hs, hd, fp_table, drug_fp)


def _tc_post_body(aggp_ref, denp_ref, dt_ref, ctlp_ref, ci_ref, cell_ref,
                  fpf_ref, wfp_ref, wout_ref, out_ref):
    agg = aggp_ref[0] + aggp_ref[1]
    den = denp_ref[0] + denp_ref[1] + jnp.float32(1e-9)
    g = agg / den
    gene = jnp.where(g > 0, g, jnp.exp(g) - 1.0)
    dt_emb = jnp.dot(dt_ref[...], gene,
                     preferred_element_type=jnp.float32,
                     precision=lax.Precision.HIGHEST)
    ci = ci_ref[...]
    oh = (ci == lax.broadcasted_iota(jnp.int32, (B, N_CELLS), 1))
    cell_emb = jnp.dot(oh.astype(jnp.float32), cell_ref[...],
                       preferred_element_type=jnp.float32,
                       precision=lax.Precision.HIGHEST)
    fp_emb = jnp.dot(fpf_ref[...], wfp_ref[...],
                     preferred_element_type=jnp.float32,
                     precision=lax.Precision.HIGHEST)
    z = jnp.maximum(ctlp_ref[...] + dt_emb + cell_emb + fp_emb, 0.0)
    out_ref[...] = jnp.dot(z, wout_ref[...],
                           preferred_element_type=jnp.float32,
                           precision=lax.Precision.HIGHEST)


def _tc_post(aggp, denp_col, drug_targets, ctl_proj, cell_idx, cell_table,
             fp_feat, W_fp, W_out):
    return pl.pallas_call(
        _tc_post_body,
        out_shape=jax.ShapeDtypeStruct((B, N_NODES), jnp.float32),
    )(aggp, denp_col, drug_targets, ctl_proj, cell_idx, cell_table,
      fp_feat, W_fp, W_out)


def kernel(ctl, drug_targets, cell_idx, drug_fp, edge_index, edge_weight,
           fp_table, node_emb, W_g, a_s, a_d, W_ctl, W_fp, cell_table, W_out):
    h, hs, hd, ctl_proj = _tc_pre(node_emb, W_g, a_s, a_d, ctl, W_ctl)

    src = edge_index[0]
    dst = edge_index[1]
    pad = E_PAD - E_EDGES
    srcp = jnp.concatenate(
        [src, jnp.zeros((pad,), jnp.int32)]).reshape(NW * NCH, K)
    dstp = jnp.concatenate(
        [dst, jnp.zeros((pad,), jnp.int32)]).reshape(NW * NCH, K)
    wp = jnp.concatenate(
        [edge_weight, jnp.zeros((pad,), jnp.float32)]).reshape(NW * NCH, K)

    aggp, denp, fp_feat = _sc_edge(
        h, srcp, dstp, wp, hs.reshape(N_NODES), hd.reshape(N_NODES),
        fp_table, drug_fp.astype(jnp.int32))

    denp_col = denp[:, :N_NODES].reshape(2, N_NODES, 1)
    out = _tc_post(aggp, denp_col, drug_targets, ctl_proj,
                   cell_idx.astype(jnp.int32).reshape(B, 1), cell_table,
                   fp_feat, W_fp, W_out)
    return out


# speedup vs baseline: 14.3726x; 14.3726x over previous
import functools

import jax
import jax.numpy as jnp
from jax import lax
from jax.experimental import pallas as pl
from jax.experimental.pallas import tpu as pltpu
from jax.experimental.pallas import tpu_sc as plsc

N_NODES = 10000
E_EDGES = 320000
D = 128
B = 64
FP_DIM = 1024
N_CELLS = 100

NW = 32
K = 128
NCH = 80
E_PAD = NW * NCH * K
DEN_PAD = 10240
AGG_PAD = 10240


def _tc_pre_body(emb_ref, wg_ref, as_ref, ad_ref, ctl_ref, wctl_ref,
                 h_ref, hs_ref, hd_ref, ctlp_ref):
    h = jnp.dot(emb_ref[...], wg_ref[...],
                preferred_element_type=jnp.float32,
                precision=lax.Precision.HIGHEST)
    h_ref[...] = h
    hs_ref[...] = jnp.sum(h * as_ref[...], axis=1, keepdims=True)
    hd_ref[...] = jnp.sum(h * ad_ref[...], axis=1, keepdims=True)
    ctlp_ref[...] = jnp.dot(ctl_ref[...], wctl_ref[...],
                            preferred_element_type=jnp.float32,
                            precision=lax.Precision.HIGHEST)


def _tc_pre(node_emb, W_g, a_s, a_d, ctl, W_ctl):
    return pl.pallas_call(
        _tc_pre_body,
        out_shape=(
            jax.ShapeDtypeStruct((N_NODES, D), jnp.float32),
            jax.ShapeDtypeStruct((N_NODES, 1), jnp.float32),
            jax.ShapeDtypeStruct((N_NODES, 1), jnp.float32),
            jax.ShapeDtypeStruct((B, D), jnp.float32),
        ),
    )(node_emb, W_g, a_s.reshape(1, D), a_d.reshape(1, D), ctl, W_ctl)


def _sc_kernel_body(h_hbm, edges_hbm, hs_hbm, hd_hbm,
                    fp8_hbm, drug_fp_hbm,
                    agg_out, den_out, fp_out,
                    hs_v, hd_v, edges_v, ee_v, rows_v,
                    zden_v, fpi_v, fpbuf_v, agg_sh, den_sh, sem):
    cid = lax.axis_index("c")
    sid = lax.axis_index("s")
    wid = cid * 16 + sid

    z16f = jnp.zeros((16,), jnp.float32)

    def _zrow(r, carry):
        for d8 in range(8):
            rows_v[r, pl.ds(16 * d8, 16)] = z16f
        return carry
    lax.fori_loop(0, K, _zrow, 0)

    def _zden(i, carry):
        zden_v[pl.ds(i * 16, 16)] = z16f
        return carry
    lax.fori_loop(0, 40, _zden, 0)

    base = sid * 640
    for k in range(5):
        pltpu.sync_copy(rows_v, agg_sh.at[pl.ds(base + 128 * k, 128)])
    pltpu.sync_copy(zden_v, den_sh.at[pl.ds(sid * 640, 640)])

    pltpu.sync_copy(hs_hbm, hs_v)
    pltpu.sync_copy(hd_hbm, hd_v)

    pltpu.sync_copy(drug_fp_hbm, fpi_v)
    gbase = wid * 16
    gv = gbase + lax.iota(jnp.int32, 16)
    q = lax.shift_right_logical(gv, 3)
    rem = lax.bitwise_and(gv, 7)
    fpi16 = plsc.load_gather(fpi_v, [q]) * 8 + rem
    pltpu.async_copy(fp8_hbm.at[fpi16], fpbuf_v, sem).wait()
    pltpu.sync_copy(fpbuf_v, fp_out.at[pl.ds(gbase, 16)])

    plsc.subcore_barrier()

    def _chunk(ch, carry):
        pltpu.sync_copy(edges_hbm.at[wid * NCH + ch], edges_v)
        pltpu.async_copy(h_hbm.at[edges_v.at[0]], rows_v, sem).wait()
        for g in range(8):
            s16 = edges_v[0, pl.ds(16 * g, 16)]
            d16 = edges_v[1, pl.ds(16 * g, 16)]
            w16 = plsc.bitcast(edges_v[2, pl.ds(16 * g, 16)], jnp.float32)
            a = plsc.load_gather(hs_v, [s16])
            b = plsc.load_gather(hd_v, [d16])
            x = a + b
            e = jnp.where(x >= 0, x, x * jnp.float32(0.2))
            ee_v[pl.ds(16 * g, 16)] = jnp.exp(e) * w16

        def _scale(g, carry2):
            for j in range(16):
                r = g * 16 + j
                sp = plsc.load_gather(
                    ee_v, [jnp.broadcast_to(r.astype(jnp.int32), (16,))])
                for d8 in range(8):
                    sl = pl.ds(16 * d8, 16)
                    rows_v[r, sl] = rows_v[r, sl] * sp
            return carry2
        lax.fori_loop(0, 8, _scale, 0)

        pltpu.sync_copy(rows_v, agg_sh.at[edges_v.at[1]], add=True)
        pltpu.sync_copy(ee_v, den_sh.at[edges_v.at[1]], add=True)
        return carry

    lax.fori_loop(0, NCH, _chunk, 0)

    plsc.subcore_barrier()

    @pl.when(sid == 0)
    def _copy_out():
        pltpu.sync_copy(agg_sh.at[pl.ds(0, N_NODES)], agg_out.at[cid])
        pltpu.sync_copy(den_sh, den_out.at[cid])


def _sc_edge(h, edges, hs, hd, fp8, drug_fp):
    mesh = plsc.VectorSubcoreMesh(core_axis_name="c", subcore_axis_name="s")
    kern = pl.kernel(
        _sc_kernel_body, mesh=mesh,
        out_type=(
            jax.ShapeDtypeStruct((2, N_NODES, D), jnp.float32),
            jax.ShapeDtypeStruct((2, DEN_PAD), jnp.float32),
            jax.ShapeDtypeStruct((B * 8, D), jnp.float32),
        ),
        scratch_types=[
            pltpu.VMEM((N_NODES,), jnp.float32),
            pltpu.VMEM((N_NODES,), jnp.float32),
            pltpu.VMEM((3, K), jnp.int32),
            pltpu.VMEM((K,), jnp.float32),
            pltpu.VMEM((K, D), jnp.float32),
            pltpu.VMEM((640,), jnp.float32),
            pltpu.VMEM((B,), jnp.int32),
            pltpu.VMEM((16, D), jnp.float32),
            pltpu.VMEM_SHARED((AGG_PAD, D), jnp.float32),
            pltpu.VMEM_SHARED((DEN_PAD,), jnp.float32),
            pltpu.SemaphoreType.DMA,
        ],
        compiler_params=pltpu.CompilerParams(needs_layout_passes=False),
    )
    return kern(h, edges, hs, hd, fp8, drug_fp)


def _tc_post_body(aggp_ref, denp_ref, dt_ref, ctlp_ref, ci_ref, cell_ref,
                  fpf_ref, wfp_ref, wout_ref, out_ref):
    agg = aggp_ref[0] + aggp_ref[1]
    den = denp_ref[0] + denp_ref[1] + jnp.float32(1e-9)
    g = agg / den
    gene = jnp.where(g > 0, g, jnp.exp(g) - 1.0)
    dt_emb = jnp.dot(dt_ref[...], gene,
                     preferred_element_type=jnp.float32,
                     precision=lax.Precision.HIGHEST)
    ci = ci_ref[...]
    oh = (ci == lax.broadcasted_iota(jnp.int32, (B, N_CELLS), 1))
    cell_emb = jnp.dot(oh.astype(jnp.float32), cell_ref[...],
                       preferred_element_type=jnp.float32,
                       precision=lax.Precision.HIGHEST)
    fp_emb = jnp.dot(fpf_ref[...], wfp_ref[...],
                     preferred_element_type=jnp.float32,
                     precision=lax.Precision.HIGHEST)
    z = jnp.maximum(ctlp_ref[...] + dt_emb + cell_emb + fp_emb, 0.0)
    out_ref[...] = jnp.dot(z, wout_ref[...],
                           preferred_element_type=jnp.float32,
                           precision=lax.Precision.HIGHEST)


def _tc_post(aggp, denp_col, drug_targets, ctl_proj, cell_idx, cell_table,
             fp_feat, W_fp, W_out):
    return pl.pallas_call(
        _tc_post_body,
        out_shape=jax.ShapeDtypeStruct((B, N_NODES), jnp.float32),
    )(aggp, denp_col, drug_targets, ctl_proj, cell_idx, cell_table,
      fp_feat, W_fp, W_out)


def kernel(ctl, drug_targets, cell_idx, drug_fp, edge_index, edge_weight,
           fp_table, node_emb, W_g, a_s, a_d, W_ctl, W_fp, cell_table, W_out):
    h, hs, hd, ctl_proj = _tc_pre(node_emb, W_g, a_s, a_d, ctl, W_ctl)

    src = edge_index[0]
    dst = edge_index[1]
    pad = E_PAD - E_EDGES
    srcp = jnp.concatenate(
        [src, jnp.zeros((pad,), jnp.int32)]).reshape(NW * NCH, 1, K)
    dstp = jnp.concatenate(
        [dst, jnp.zeros((pad,), jnp.int32)]).reshape(NW * NCH, 1, K)
    wp = lax.bitcast_convert_type(
        jnp.concatenate([edge_weight, jnp.zeros((pad,), jnp.float32)]),
        jnp.int32).reshape(NW * NCH, 1, K)
    edges = jnp.concatenate([srcp, dstp, wp], axis=1)

    aggp, denp, fp8_rows = _sc_edge(
        h, edges, hs.reshape(N_NODES), hd.reshape(N_NODES),
        fp_table.reshape(2000 * 8, D), drug_fp.astype(jnp.int32))

    fp_feat = fp8_rows.reshape(B, FP_DIM)
    denp_col = denp[:, :N_NODES].reshape(2, N_NODES, 1)
    out = _tc_post(aggp, denp_col, drug_targets, ctl_proj,
                   cell_idx.astype(jnp.int32).reshape(B, 1), cell_table,
                   fp_feat, W_fp, W_out)
    return out
